# Initial kernel scaffold; baseline (speedup 1.0000x reference)
import jax
import jax.numpy as jnp
from jax.experimental import pallas as pl


def kernel(signal, table):
    raise NotImplementedError("write your pallas kernel here")


# speedup vs baseline: 1.0760x; 1.0760x over previous
import functools

import jax
import jax.numpy as jnp
from jax import lax
from jax.experimental import pallas as pl
from jax.experimental.pallas import tpu as pltpu
from jax.experimental.pallas import tpu_sc as plsc

_INFO = plsc.get_sparse_core_info()
_NC = _INFO.num_cores
_NS = _INFO.num_subcores
_NW = _NC * _NS

_B = 1024 * 20
_D = 2048
_V = 38
_BPW = _B // _NW
_C = 16
_NCH = _BPW // _C


def _body(table_hbm, idx_hbm, out_hbm, idx_v, rows_v, sem0, sem1):
    wid = lax.axis_index("s") * _NC + lax.axis_index("c")
    base = wid * _BPW
    pltpu.sync_copy(idx_hbm.at[pl.ds(base, _BPW)], idx_v)

    sems = (sem0, sem1)

    def gather_start(c, b):
        pltpu.async_copy(
            table_hbm.at[idx_v.at[pl.ds(c * _C, _C)]], rows_v.at[b], sems[b]
        )

    def gather_wait(c, b):
        pltpu.make_async_copy(
            table_hbm.at[idx_v.at[pl.ds(c * _C, _C)]], rows_v.at[b], sems[b]
        ).wait()

    gather_start(0, 0)
    gather_start(1, 1)

    @pl.loop(0, _NCH, step=2)
    def _(g):
        for b in range(2):
            c = g + b
            gather_wait(c, b)
            pltpu.sync_copy(rows_v.at[b], out_hbm.at[pl.ds(base + c * _C, _C)])

            @pl.when(c + 2 < _NCH)
            def _():
                gather_start(c + 2, b)


def kernel(signal, table):
    idx = signal.reshape(-1).astype(jnp.int32)
    mesh = plsc.VectorSubcoreMesh(core_axis_name="c", subcore_axis_name="s")
    run = pl.kernel(
        _body,
        mesh=mesh,
        out_type=jax.ShapeDtypeStruct((_B, _D), jnp.float32),
        scratch_types=[
            pltpu.VMEM((_BPW,), jnp.int32),
            pltpu.VMEM((2, _C, _D), jnp.float32),
            pltpu.SemaphoreType.DMA,
            pltpu.SemaphoreType.DMA,
        ],
    )
    out = run(table, idx)
    return out.reshape(signal.shape + (_D,))


# speedup vs baseline: 1.4663x; 1.3627x over previous
import jax
import jax.numpy as jnp
from jax import lax
from jax.experimental import pallas as pl
from jax.experimental.pallas import tpu as pltpu
from jax.experimental.pallas import tpu_sc as plsc

_INFO = plsc.get_sparse_core_info()
_NC = _INFO.num_cores
_NS = _INFO.num_subcores
_NW = _NC * _NS

_B = 1024 * 20
_D = 2048
_V = 38
_BPW = _B // _NW
_K = 16


def _body(table_hbm, idx_hbm, out_hbm, table_v, idx_v, *sems):
    wid = lax.axis_index("s") * _NC + lax.axis_index("c")
    base = wid * _BPW
    pltpu.sync_copy(idx_hbm.at[pl.ds(base, _BPW)], idx_v)
    pltpu.sync_copy(table_hbm, table_v)

    def row_start(i, v, b):
        pltpu.async_copy(table_v.at[v], out_hbm.at[base + i], sems[b])

    def row_wait(b):
        pltpu.make_async_copy(table_v.at[0], out_hbm.at[base], sems[b]).wait()

    @pl.loop(0, _BPW, step=_K)
    def _(g):
        @pl.when(g > 0)
        def _():
            for b in range(_K):
                row_wait(b)

        vals = idx_v[pl.ds(g, _K)]
        for b in range(_K):
            row_start(g + b, vals[b], b)

    for b in range(_K):
        row_wait(b)


def kernel(signal, table):
    idx = signal.reshape(-1).astype(jnp.int32)
    mesh = plsc.VectorSubcoreMesh(core_axis_name="c", subcore_axis_name="s")
    run = pl.kernel(
        _body,
        mesh=mesh,
        out_type=jax.ShapeDtypeStruct((_B, _D), jnp.float32),
        scratch_types=[
            pltpu.VMEM((_V, _D), jnp.float32),
            pltpu.VMEM((_BPW,), jnp.int32),
        ]
        + [pltpu.SemaphoreType.DMA] * _K,
    )
    out = run(table, idx)
    return out.reshape(signal.shape + (_D,))
